# baseline (speedup 1.0000x reference)
import functools

import jax
import jax.numpy as jnp
from jax import lax
from jax.experimental import pallas as pl
from jax.experimental.pallas import tpu as pltpu
from jax.experimental.pallas import tpu_sc as plsc

D_MODEL = 256
B = 16
T = 2048

N = B * T
NW = 32
TW = T // NW
LANES = 16
NVEC = D_MODEL // LANES
NBUF = 6
DEPTH = 4

_mesh = plsc.VectorSubcoreMesh(core_axis_name="c", subcore_axis_name="s")


@functools.partial(
    pl.kernel,
    mesh=_mesh,
    out_type=jax.ShapeDtypeStruct((N, D_MODEL), jnp.float32),
    scratch_types=[
        pltpu.VMEM((B, TW), jnp.int32),
        pltpu.VMEM((TW, D_MODEL), jnp.float32),
    ]
    + [pltpu.VMEM((TW, D_MODEL), jnp.float32) for _ in range(NBUF)]
    + [pltpu.SemaphoreType.DMA for _ in range(2 * NBUF + 1)],
)
def _emb_lookup(x_hbm, tok_hbm, pos_hbm, out_hbm, idx_v, pos_v, *rest):
    bufs = list(rest[:NBUF])
    gsems = list(rest[NBUF : 2 * NBUF])
    osems = list(rest[2 * NBUF : 3 * NBUF])
    ssem = rest[3 * NBUF]

    wid = lax.axis_index("s") * 2 + lax.axis_index("c")
    t0 = wid * TW

    staged = [
        pltpu.async_copy(x_hbm.at[pl.ds(b * T + t0, TW)], idx_v.at[b], ssem)
        for b in range(B)
    ]
    staged.append(pltpu.async_copy(pos_hbm.at[pl.ds(t0, TW)], pos_v, ssem))
    for d in staged:
        d.wait()

    def gather(b):
        s = b % NBUF
        return pltpu.async_copy(tok_hbm.at[idx_v.at[b]], bufs[s], gsems[s])

    gd = {}
    od = {}
    for b in range(DEPTH):
        gd[b % NBUF] = gather(b)

    for b in range(B):
        s = b % NBUF
        gd.pop(s).wait()

        buf = bufs[s]

        @plsc.parallel_loop(0, TW, step=1, unroll=2)
        def add_row(r):
            for j in range(NVEC):
                sl = pl.ds(j * LANES, LANES)
                plsc.addupdate(buf.at[r, sl], pos_v[r, sl])

        od[s] = pltpu.async_copy(buf, out_hbm.at[pl.ds(b * T + t0, TW)], osems[s])

        nb = b + DEPTH
        if nb < B:
            ns = nb % NBUF
            if ns in od:
                od.pop(ns).wait()
            gd[ns] = gather(nb)

    for s in sorted(od):
        od.pop(s).wait()


def kernel(x, token_table, pos_table):
    xf = x.reshape(-1).astype(jnp.int32)
    out = _emb_lookup(xf, token_table, pos_table)
    return out.reshape(B, T, D_MODEL)


# speedup vs baseline: 1.0329x; 1.0329x over previous
import functools

import jax
import jax.numpy as jnp
from jax import lax
from jax.experimental import pallas as pl
from jax.experimental.pallas import tpu as pltpu
from jax.experimental.pallas import tpu_sc as plsc

D_MODEL = 256
B = 16
T = 2048

N = B * T
NW = 32
TW = T // NW
LANES = 16
NVEC = D_MODEL // LANES
NBUF = 6
DEPTH = 4

_mesh = plsc.VectorSubcoreMesh(core_axis_name="c", subcore_axis_name="s")


@functools.partial(
    pl.kernel,
    mesh=_mesh,
    out_type=jax.ShapeDtypeStruct((N, D_MODEL), jnp.float32),
    scratch_types=[
        pltpu.VMEM((B, 2 * TW), jnp.int32),
        pltpu.VMEM((TW, D_MODEL), jnp.float32),
    ]
    + [pltpu.VMEM((TW, D_MODEL), jnp.float32) for _ in range(NBUF)]
    + [pltpu.SemaphoreType.DMA for _ in range(2 * NBUF + 1)],
)
def _emb_lookup(x_hbm, tok_hbm, pos_hbm, out_hbm, idx_v, pos_v, *rest):
    bufs = list(rest[:NBUF])
    gsems = list(rest[NBUF : 2 * NBUF])
    osems = list(rest[2 * NBUF : 3 * NBUF])
    ssem = rest[3 * NBUF]

    wid = lax.axis_index("s") * 2 + lax.axis_index("c")
    t0 = wid * TW
    ta = pl.multiple_of((wid // 2) * (2 * TW), 2 * TW)
    off = pl.multiple_of((wid % 2) * TW, TW)

    staged = [
        pltpu.async_copy(x_hbm.at[:, pl.ds(ta, 2 * TW)], idx_v, ssem),
        pltpu.async_copy(pos_hbm.at[pl.ds(t0, TW)], pos_v, ssem),
    ]
    for d in staged:
        d.wait()

    def gather(b):
        s = b % NBUF
        return pltpu.async_copy(
            tok_hbm.at[idx_v.at[b, pl.ds(off, TW)]], bufs[s], gsems[s]
        )

    gd = {}
    od = {}
    for b in range(DEPTH):
        gd[b % NBUF] = gather(b)

    for b in range(B):
        s = b % NBUF
        gd.pop(s).wait()

        buf = bufs[s]

        def add_row(r, carry):
            for j in range(NVEC):
                sl = pl.ds(j * LANES, LANES)
                plsc.addupdate(buf.at[r, sl], pos_v[r, sl])
            return carry

        lax.fori_loop(0, TW, add_row, 0)

        od[s] = pltpu.async_copy(buf, out_hbm.at[pl.ds(b * T + t0, TW)], osems[s])

        nb = b + DEPTH
        if nb < B:
            ns = nb % NBUF
            if ns in od:
                od.pop(ns).wait()
            gd[ns] = gather(nb)

    for s in sorted(od):
        od.pop(s).wait()


def kernel(x, token_table, pos_table):
    out = _emb_lookup(x.astype(jnp.int32), token_table, pos_table)
    return out.reshape(B, T, D_MODEL)
